# baseline (speedup 1.0000x reference)
import jax
import jax.numpy as jnp
from jax import lax
from jax.experimental import pallas as pl
from jax.experimental.pallas import tpu as pltpu
from jax.experimental.pallas import tpu_sc as plsc

D = 1024
F = 512
SF = 1024
E = 16
G = 4
GSZ = E // G
T = 2048
K = 2
SCALE = 2.5
BLK = 128
SBLK = 256
NBMAX = E + (K * T) // BLK
NPAD = NBMAX * BLK
NEG = -1e30

NC, NS = 2, 16
NW = NC * NS
CH = T // NW

_HI = jax.lax.Precision.HIGHEST


def _tcol(row):
    n = row.shape[1]
    r = lax.broadcasted_iota(jnp.int32, (n, n), 0)
    c = lax.broadcasted_iota(jnp.int32, (n, n), 1)
    eye = (r == c).astype(jnp.float32)
    return lax.dot_general(eye, row, (((1,), (1,)), ((), ())), precision=_HI)


def _meta_body(idx_ref, w_ref, wcol_ref, dcol_ref, meta_ref):
    lane = lax.broadcasted_iota(jnp.int32, (T, E), 1)
    m0 = lane == idx_ref[:, 0:1]
    m1 = lane == idx_ref[:, 1:2]
    w0 = w_ref[:, 0:1]
    w1 = w_ref[:, 1:2]

    onehot = (m0 | m1).astype(jnp.float32)

    c = onehot
    sh = 1
    while sh < T:
        c = c + jnp.concatenate([jnp.zeros((sh, E), jnp.float32), c[:-sh]], axis=0)
        sh *= 2
    pos = c - onehot
    counts = c[T - 1:T, :]

    nb = jnp.floor((counts + (BLK - 1)) * (1.0 / BLK))
    up = lax.broadcasted_iota(jnp.int32, (E, E), 0)
    lo = lax.broadcasted_iota(jnp.int32, (E, E), 1)
    strict = (up < lo).astype(jnp.float32)
    bstart = lax.dot_general(nb, strict, (((1,), (0,)), ((), ())), precision=_HI)
    nbtot = jnp.sum(nb, axis=1, keepdims=True)

    dstv = pos + BLK * bstart
    d0 = jnp.sum(jnp.where(m0, dstv, 0.0), axis=1, keepdims=True)
    d1 = jnp.sum(jnp.where(m1, dstv, 0.0), axis=1, keepdims=True)

    l128 = lax.broadcasted_iota(jnp.int32, (T, 128), 1)
    wcol_ref[...] = (jnp.where(l128 == 0, w0, 0.0)
                     + jnp.where(l128 == 1, w1, 0.0))
    dcol_ref[...] = (jnp.where(l128 == 0, d0, 0.0)
                     + jnp.where(l128 == 1, d1, 0.0)).astype(jnp.int32)

    bi = lax.broadcasted_iota(jnp.int32, (1, 128), 1).astype(jnp.float32)
    iclamp = jnp.minimum(bi, nbtot - 1.0)
    bstart_col = _tcol(bstart)
    be = jnp.sum((iclamp >= bstart_col).astype(jnp.float32),
                 axis=0, keepdims=True) - 1.0
    meta_ref[...] = jnp.concatenate(
        [be, iclamp, jnp.zeros((6, 128), jnp.float32)], axis=0).astype(jnp.int32)


def _meta(idx, w):
    return pl.pallas_call(
        _meta_body,
        out_shape=[
            jax.ShapeDtypeStruct((T, 128), jnp.float32),
            jax.ShapeDtypeStruct((T, 128), jnp.int32),
            jax.ShapeDtypeStruct((8, 128), jnp.int32),
        ],
    )(idx, w)


def _router_body(x_ref, gw_ref, gb_ref, wcol_ref, dcol_ref, meta_ref):
    x = x_ref[...]
    gw = gw_ref[...]
    gb = gb_ref[...]
    logits = lax.dot_general(x, gw, (((1,), (1,)), ((), ())), precision=_HI)
    scores = jax.nn.sigmoid(logits)
    s4 = scores + gb

    lane = lax.broadcasted_iota(jnp.int32, (T, E), 1)
    gid = lane // GSZ

    gs16 = jnp.zeros((T, E), jnp.float32)
    for g in range(G):
        m = gid == g
        a = jnp.where(m, s4, NEG)
        v1 = jnp.max(a, axis=1, keepdims=True)
        idx1 = jnp.min(jnp.where(a == v1, lane, E + 1), axis=1, keepdims=True)
        a2 = jnp.where(lane == idx1, NEG, a)
        v2 = jnp.max(a2, axis=1, keepdims=True)
        gs16 = jnp.where(m, v1 + v2, gs16)

    gmax1 = jnp.max(gs16, axis=1, keepdims=True)
    g1 = jnp.min(jnp.where(gs16 == gmax1, gid, G + 1), axis=1, keepdims=True)
    rest = jnp.where(gid == g1, NEG, gs16)
    gmax2 = jnp.max(rest, axis=1, keepdims=True)
    g2 = jnp.min(jnp.where(rest == gmax2, gid, G + 1), axis=1, keepdims=True)
    allowed = (gid == g1) | (gid == g2)

    ms = jnp.where(allowed, s4, NEG)
    v1 = jnp.max(ms, axis=1, keepdims=True)
    e1 = jnp.min(jnp.where(ms == v1, lane, E + 1), axis=1, keepdims=True)
    m0 = lane == e1
    ms2 = jnp.where(m0, NEG, ms)
    v2 = jnp.max(ms2, axis=1, keepdims=True)
    e2 = jnp.min(jnp.where(ms2 == v2, lane, E + 1), axis=1, keepdims=True)
    m1 = lane == e2

    w0 = jnp.sum(jnp.where(m0, scores, 0.0), axis=1, keepdims=True)
    w1 = jnp.sum(jnp.where(m1, scores, 0.0), axis=1, keepdims=True)
    den = w0 + w1 + 1e-6
    w0 = w0 / den * SCALE
    w1 = w1 / den * SCALE

    onehot = (m0 | m1).astype(jnp.float32)

    c = onehot
    sh = 1
    while sh < T:
        c = c + jnp.concatenate([jnp.zeros((sh, E), jnp.float32), c[:-sh]], axis=0)
        sh *= 2
    pos = c - onehot
    counts = c[T - 1:T, :]

    nb = jnp.floor((counts + (BLK - 1)) * (1.0 / BLK))
    up = lax.broadcasted_iota(jnp.int32, (E, E), 0)
    lo = lax.broadcasted_iota(jnp.int32, (E, E), 1)
    strict = (up < lo).astype(jnp.float32)
    bstart = lax.dot_general(nb, strict, (((1,), (0,)), ((), ())), precision=_HI)
    nbtot = jnp.sum(nb, axis=1, keepdims=True)

    dstv = pos + BLK * bstart
    d0 = jnp.sum(jnp.where(m0, dstv, 0.0), axis=1, keepdims=True)
    d1 = jnp.sum(jnp.where(m1, dstv, 0.0), axis=1, keepdims=True)

    l128 = lax.broadcasted_iota(jnp.int32, (T, 128), 1)
    wcol_ref[...] = (jnp.where(l128 == 0, w0, 0.0)
                     + jnp.where(l128 == 1, w1, 0.0))
    dcol_ref[...] = (jnp.where(l128 == 0, d0, 0.0)
                     + jnp.where(l128 == 1, d1, 0.0)).astype(jnp.int32)

    bi = lax.broadcasted_iota(jnp.int32, (1, 128), 1).astype(jnp.float32)
    iclamp = jnp.minimum(bi, nbtot - 1.0)
    bstart_col = _tcol(bstart)
    be = jnp.sum((iclamp >= bstart_col).astype(jnp.float32),
                 axis=0, keepdims=True) - 1.0
    meta_ref[...] = jnp.concatenate(
        [be, iclamp, jnp.zeros((6, 128), jnp.float32)], axis=0).astype(jnp.int32)


def _router(x, gw, gb):
    return pl.pallas_call(
        _router_body,
        out_shape=[
            jax.ShapeDtypeStruct((T, 128), jnp.float32),
            jax.ShapeDtypeStruct((T, 128), jnp.int32),
            jax.ShapeDtypeStruct((8, 128), jnp.int32),
        ],
    )(x, gw, gb)


def _dispatch_body(x_hbm, d0_hbm, d1_hbm, xs_hbm, idx_v, rows_v, sem):
    wid = lax.axis_index("s") * NC + lax.axis_index("c")
    base = wid * CH
    pltpu.sync_copy(x_hbm.at[pl.ds(base, CH)], rows_v)
    pltpu.sync_copy(d0_hbm.at[pl.ds(base, CH)], idx_v)
    pltpu.async_copy(rows_v, xs_hbm.at[idx_v], sem).wait()
    pltpu.sync_copy(d1_hbm.at[pl.ds(base, CH)], idx_v)
    pltpu.async_copy(rows_v, xs_hbm.at[idx_v], sem).wait()


def _dispatch(x, d0, d1):
    f = pl.kernel(
        _dispatch_body,
        mesh=plsc.VectorSubcoreMesh(core_axis_name="c", subcore_axis_name="s"),
        out_type=jax.ShapeDtypeStruct((NPAD, D), jnp.float32),
        scratch_types=[
            pltpu.VMEM((CH,), jnp.int32),
            pltpu.VMEM((CH, D), jnp.float32),
            pltpu.SemaphoreType.DMA,
        ],
    )
    return f(x, d0, d1)


def _gather_body(ys_hbm, d0_hbm, d1_hbm, r0_hbm, r1_hbm, idx_v, rows_v, sem):
    wid = lax.axis_index("s") * NC + lax.axis_index("c")
    base = wid * CH
    pltpu.sync_copy(d0_hbm.at[pl.ds(base, CH)], idx_v)
    pltpu.async_copy(ys_hbm.at[idx_v], rows_v, sem).wait()
    pltpu.sync_copy(rows_v, r0_hbm.at[pl.ds(base, CH)])
    pltpu.sync_copy(d1_hbm.at[pl.ds(base, CH)], idx_v)
    pltpu.async_copy(ys_hbm.at[idx_v], rows_v, sem).wait()
    pltpu.sync_copy(rows_v, r1_hbm.at[pl.ds(base, CH)])


def _gather(ys, d0, d1):
    f = pl.kernel(
        _gather_body,
        mesh=plsc.VectorSubcoreMesh(core_axis_name="c", subcore_axis_name="s"),
        out_type=[
            jax.ShapeDtypeStruct((T, D), jnp.float32),
            jax.ShapeDtypeStruct((T, D), jnp.float32),
        ],
        scratch_types=[
            pltpu.VMEM((CH,), jnp.int32),
            pltpu.VMEM((CH, D), jnp.float32),
            pltpu.SemaphoreType.DMA,
        ],
    )
    return f(ys, d0, d1)


def _gemm_body(be_ref, xr_ref, xs_ref, w1_ref, w3_ref, w2_ref, ys_ref):
    xb = xs_ref[...]
    h1 = lax.dot_general(xb, w1_ref[0], (((1,), (1,)), ((), ())),
                         preferred_element_type=jnp.float32)
    h3 = lax.dot_general(xb, w3_ref[0], (((1,), (1,)), ((), ())),
                         preferred_element_type=jnp.float32)
    g = (h1 * jax.nn.sigmoid(h1) * h3).astype(jnp.bfloat16)
    ys_ref[...] = lax.dot_general(g, w2_ref[0], (((1,), (1,)), ((), ())),
                                  preferred_element_type=jnp.float32)


def _gemm(be, xr, xs, w1, w3, w2):
    grid_spec = pltpu.PrefetchScalarGridSpec(
        num_scalar_prefetch=2,
        grid=(NBMAX,),
        in_specs=[
            pl.BlockSpec((BLK, D), lambda i, be, xr: (xr[i], 0)),
            pl.BlockSpec((1, F, D), lambda i, be, xr: (be[i], 0, 0)),
            pl.BlockSpec((1, F, D), lambda i, be, xr: (be[i], 0, 0)),
            pl.BlockSpec((1, D, F), lambda i, be, xr: (be[i], 0, 0)),
        ],
        out_specs=pl.BlockSpec((BLK, D), lambda i, be, xr: (xr[i], 0)),
    )
    return pl.pallas_call(
        _gemm_body,
        grid_spec=grid_spec,
        out_shape=jax.ShapeDtypeStruct((NPAD, D), jnp.float32),
    )(be, xr, xs, w1, w3, w2)


def _shared_body(x_ref, sw1_ref, sw3_ref, sw2_ref, r0_ref, r1_ref, wc_ref, o_ref):
    xb = x_ref[...]
    h1 = lax.dot_general(xb, sw1_ref[...], (((1,), (1,)), ((), ())),
                         preferred_element_type=jnp.float32)
    h3 = lax.dot_general(xb, sw3_ref[...], (((1,), (1,)), ((), ())),
                         preferred_element_type=jnp.float32)
    g = (h1 * jax.nn.sigmoid(h1) * h3).astype(jnp.bfloat16)
    y = lax.dot_general(g, sw2_ref[...], (((1,), (1,)), ((), ())),
                        preferred_element_type=jnp.float32)
    o_ref[...] = (y + wc_ref[:, 0:1] * r0_ref[...]
                  + wc_ref[:, 1:2] * r1_ref[...])


def _shared(x, sw1, sw3, sw2, r0, r1, wcol):
    nblk = T // SBLK
    return pl.pallas_call(
        _shared_body,
        grid=(nblk,),
        in_specs=[
            pl.BlockSpec((SBLK, D), lambda i: (i, 0)),
            pl.BlockSpec((SF, D), lambda i: (0, 0)),
            pl.BlockSpec((SF, D), lambda i: (0, 0)),
            pl.BlockSpec((D, SF), lambda i: (0, 0)),
            pl.BlockSpec((SBLK, D), lambda i: (i, 0)),
            pl.BlockSpec((SBLK, D), lambda i: (i, 0)),
            pl.BlockSpec((SBLK, 128), lambda i: (i, 0)),
        ],
        out_specs=pl.BlockSpec((SBLK, D), lambda i: (i, 0)),
        out_shape=jax.ShapeDtypeStruct((T, D), jnp.float32),
    )(x, sw1, sw3, sw2, r0, r1, wcol)


def kernel(hidden_states, gate_weight, gate_bias, w1, w3, w2, sw1, sw3, sw2):
    shape = hidden_states.shape
    x = hidden_states.reshape(-1, D)

    logits = x @ gate_weight.T
    scores = jax.nn.sigmoid(logits)
    scores_for_topk = scores + gate_bias
    scores_view = scores_for_topk.reshape(T, G, -1)
    group_scores = lax.top_k(scores_view, 2)[0].sum(axis=-1)
    group_idx = lax.top_k(group_scores, 2)[1]
    mask = jnp.ones((T, G), dtype=bool)
    mask = mask.at[jnp.arange(T)[:, None], group_idx].set(False)
    masked = jnp.where(mask[:, :, None], -jnp.inf, scores_view).reshape(T, E)
    _, indices = lax.top_k(masked, K)
    weights = jnp.take_along_axis(scores, indices, axis=1)
    weights = weights / (weights.sum(axis=-1, keepdims=True) + 1e-6)
    weights = weights * SCALE

    wcol, dcol, meta = _meta(indices.astype(jnp.int32), weights)
    be = meta[0, :NBMAX]
    xr = meta[1, :NBMAX]
    d0 = dcol[:, 0]
    d1 = dcol[:, 1]

    xs = _dispatch(x, d0, d1)
    ys = _gemm(be, xr, xs, w1, w3, w2)
    r0, r1 = _gather(ys, d0, d1)
    out = _shared(x, sw1, sw3, sw2, r0, r1, wcol)
    return out.reshape(shape)


# speedup vs baseline: 1.3875x; 1.3875x over previous
import jax
import jax.numpy as jnp
from jax import lax
from jax.experimental import pallas as pl
from jax.experimental.pallas import tpu as pltpu
from jax.experimental.pallas import tpu_sc as plsc

D = 1024
F = 512
SF = 1024
E = 16
G = 4
GSZ = E // G
T = 2048
K = 2
SCALE = 2.5
BLK = 256
SBLK = 256
NBMAX = E + (K * T) // BLK
NPAD = NBMAX * BLK
NEG = -1e30

NC, NS = 2, 16
NW = NC * NS
CH = T // NW

_HI = jax.lax.Precision.HIGHEST


def _tcol(row):
    n = row.shape[1]
    r = lax.broadcasted_iota(jnp.int32, (n, n), 0)
    c = lax.broadcasted_iota(jnp.int32, (n, n), 1)
    eye = (r == c).astype(jnp.float32)
    return lax.dot_general(eye, row, (((1,), (1,)), ((), ())), precision=_HI)


def _meta_body(idx_ref, w_ref, wcol_ref, dcol_ref, meta_ref):
    lane = lax.broadcasted_iota(jnp.int32, (T, E), 1)
    m0 = lane == idx_ref[:, 0:1]
    m1 = lane == idx_ref[:, 1:2]
    w0 = w_ref[:, 0:1]
    w1 = w_ref[:, 1:2]

    onehot = (m0 | m1).astype(jnp.float32)

    c = onehot
    sh = 1
    while sh < T:
        c = c + jnp.concatenate([jnp.zeros((sh, E), jnp.float32), c[:-sh]], axis=0)
        sh *= 2
    pos = c - onehot
    counts = c[T - 1:T, :]

    nb = jnp.floor((counts + (BLK - 1)) * (1.0 / BLK))
    up = lax.broadcasted_iota(jnp.int32, (E, E), 0)
    lo = lax.broadcasted_iota(jnp.int32, (E, E), 1)
    strict = (up < lo).astype(jnp.float32)
    bstart = lax.dot_general(nb, strict, (((1,), (0,)), ((), ())), precision=_HI)
    nbtot = jnp.sum(nb, axis=1, keepdims=True)

    dstv = pos + BLK * bstart
    d0 = jnp.sum(jnp.where(m0, dstv, 0.0), axis=1, keepdims=True)
    d1 = jnp.sum(jnp.where(m1, dstv, 0.0), axis=1, keepdims=True)

    l128 = lax.broadcasted_iota(jnp.int32, (T, 128), 1)
    wcol_ref[...] = (jnp.where(l128 == 0, w0, 0.0)
                     + jnp.where(l128 == 1, w1, 0.0))
    dcol_ref[...] = (jnp.where(l128 == 0, d0, 0.0)
                     + jnp.where(l128 == 1, d1, 0.0)).astype(jnp.int32)

    bi = lax.broadcasted_iota(jnp.int32, (1, 128), 1).astype(jnp.float32)
    iclamp = jnp.minimum(bi, nbtot - 1.0)
    bstart_col = _tcol(bstart)
    be = jnp.sum((iclamp >= bstart_col).astype(jnp.float32),
                 axis=0, keepdims=True) - 1.0
    meta_ref[...] = jnp.concatenate(
        [be, iclamp, jnp.zeros((6, 128), jnp.float32)], axis=0).astype(jnp.int32)


def _meta(idx, w):
    return pl.pallas_call(
        _meta_body,
        out_shape=[
            jax.ShapeDtypeStruct((T, 128), jnp.float32),
            jax.ShapeDtypeStruct((T, 128), jnp.int32),
            jax.ShapeDtypeStruct((8, 128), jnp.int32),
        ],
    )(idx, w)


def _router_body(x_ref, gw_ref, gb_ref, wcol_ref, dcol_ref, meta_ref):
    x = x_ref[...]
    gw = gw_ref[...]
    gb = gb_ref[...]
    logits = lax.dot_general(x, gw, (((1,), (1,)), ((), ())), precision=_HI)
    scores = jax.nn.sigmoid(logits)
    s4 = scores + gb

    lane = lax.broadcasted_iota(jnp.int32, (T, E), 1)
    gid = lane // GSZ

    gs16 = jnp.zeros((T, E), jnp.float32)
    for g in range(G):
        m = gid == g
        a = jnp.where(m, s4, NEG)
        v1 = jnp.max(a, axis=1, keepdims=True)
        idx1 = jnp.min(jnp.where(a == v1, lane, E + 1), axis=1, keepdims=True)
        a2 = jnp.where(lane == idx1, NEG, a)
        v2 = jnp.max(a2, axis=1, keepdims=True)
        gs16 = jnp.where(m, v1 + v2, gs16)

    gmax1 = jnp.max(gs16, axis=1, keepdims=True)
    g1 = jnp.min(jnp.where(gs16 == gmax1, gid, G + 1), axis=1, keepdims=True)
    rest = jnp.where(gid == g1, NEG, gs16)
    gmax2 = jnp.max(rest, axis=1, keepdims=True)
    g2 = jnp.min(jnp.where(rest == gmax2, gid, G + 1), axis=1, keepdims=True)
    allowed = (gid == g1) | (gid == g2)

    ms = jnp.where(allowed, s4, NEG)
    v1 = jnp.max(ms, axis=1, keepdims=True)
    e1 = jnp.min(jnp.where(ms == v1, lane, E + 1), axis=1, keepdims=True)
    m0 = lane == e1
    ms2 = jnp.where(m0, NEG, ms)
    v2 = jnp.max(ms2, axis=1, keepdims=True)
    e2 = jnp.min(jnp.where(ms2 == v2, lane, E + 1), axis=1, keepdims=True)
    m1 = lane == e2

    w0 = jnp.sum(jnp.where(m0, scores, 0.0), axis=1, keepdims=True)
    w1 = jnp.sum(jnp.where(m1, scores, 0.0), axis=1, keepdims=True)
    den = w0 + w1 + 1e-6
    w0 = w0 / den * SCALE
    w1 = w1 / den * SCALE

    onehot = (m0 | m1).astype(jnp.float32)

    c = onehot
    sh = 1
    while sh < T:
        c = c + jnp.concatenate([jnp.zeros((sh, E), jnp.float32), c[:-sh]], axis=0)
        sh *= 2
    pos = c - onehot
    counts = c[T - 1:T, :]

    nb = jnp.floor((counts + (BLK - 1)) * (1.0 / BLK))
    up = lax.broadcasted_iota(jnp.int32, (E, E), 0)
    lo = lax.broadcasted_iota(jnp.int32, (E, E), 1)
    strict = (up < lo).astype(jnp.float32)
    bstart = lax.dot_general(nb, strict, (((1,), (0,)), ((), ())), precision=_HI)
    nbtot = jnp.sum(nb, axis=1, keepdims=True)

    dstv = pos + BLK * bstart
    d0 = jnp.sum(jnp.where(m0, dstv, 0.0), axis=1, keepdims=True)
    d1 = jnp.sum(jnp.where(m1, dstv, 0.0), axis=1, keepdims=True)

    l128 = lax.broadcasted_iota(jnp.int32, (T, 128), 1)
    wcol_ref[...] = (jnp.where(l128 == 0, w0, 0.0)
                     + jnp.where(l128 == 1, w1, 0.0))
    dcol_ref[...] = (jnp.where(l128 == 0, d0, 0.0)
                     + jnp.where(l128 == 1, d1, 0.0)).astype(jnp.int32)

    bi = lax.broadcasted_iota(jnp.int32, (1, 128), 1).astype(jnp.float32)
    iclamp = jnp.minimum(bi, nbtot - 1.0)
    bstart_col = _tcol(bstart)
    be = jnp.sum((iclamp >= bstart_col).astype(jnp.float32),
                 axis=0, keepdims=True) - 1.0
    meta_ref[...] = jnp.concatenate(
        [be, iclamp, jnp.zeros((6, 128), jnp.float32)], axis=0).astype(jnp.int32)


def _router(x, gw, gb):
    return pl.pallas_call(
        _router_body,
        out_shape=[
            jax.ShapeDtypeStruct((T, 128), jnp.float32),
            jax.ShapeDtypeStruct((T, 128), jnp.int32),
            jax.ShapeDtypeStruct((8, 128), jnp.int32),
        ],
    )(x, gw, gb)


def _dispatch_body(x_hbm, d0_hbm, d1_hbm, xs_hbm, idx_v, rows_v, sem):
    wid = lax.axis_index("s") * NC + lax.axis_index("c")
    base = wid * CH
    pltpu.sync_copy(x_hbm.at[pl.ds(base, CH)], rows_v)
    pltpu.sync_copy(d0_hbm.at[pl.ds(base, CH)], idx_v)
    pltpu.async_copy(rows_v, xs_hbm.at[idx_v], sem).wait()
    pltpu.sync_copy(d1_hbm.at[pl.ds(base, CH)], idx_v)
    pltpu.async_copy(rows_v, xs_hbm.at[idx_v], sem).wait()


def _dispatch(x, d0, d1):
    f = pl.kernel(
        _dispatch_body,
        mesh=plsc.VectorSubcoreMesh(core_axis_name="c", subcore_axis_name="s"),
        out_type=jax.ShapeDtypeStruct((NPAD, D), jnp.float32),
        scratch_types=[
            pltpu.VMEM((CH,), jnp.int32),
            pltpu.VMEM((CH, D), jnp.float32),
            pltpu.SemaphoreType.DMA,
        ],
    )
    return f(x, d0, d1)


def _gather_body(ys_hbm, d0_hbm, d1_hbm, r0_hbm, r1_hbm, idx_v, rows_v, sem):
    wid = lax.axis_index("s") * NC + lax.axis_index("c")
    base = wid * CH
    pltpu.sync_copy(d0_hbm.at[pl.ds(base, CH)], idx_v)
    pltpu.async_copy(ys_hbm.at[idx_v], rows_v, sem).wait()
    pltpu.sync_copy(rows_v, r0_hbm.at[pl.ds(base, CH)])
    pltpu.sync_copy(d1_hbm.at[pl.ds(base, CH)], idx_v)
    pltpu.async_copy(ys_hbm.at[idx_v], rows_v, sem).wait()
    pltpu.sync_copy(rows_v, r1_hbm.at[pl.ds(base, CH)])


def _gather(ys, d0, d1):
    f = pl.kernel(
        _gather_body,
        mesh=plsc.VectorSubcoreMesh(core_axis_name="c", subcore_axis_name="s"),
        out_type=[
            jax.ShapeDtypeStruct((T, D), jnp.float32),
            jax.ShapeDtypeStruct((T, D), jnp.float32),
        ],
        scratch_types=[
            pltpu.VMEM((CH,), jnp.int32),
            pltpu.VMEM((CH, D), jnp.float32),
            pltpu.SemaphoreType.DMA,
        ],
    )
    return f(ys, d0, d1)


def _gemm_body(be_ref, xr_ref, xs_ref, w1_ref, w3_ref, w2_ref, ys_ref):
    xb = xs_ref[...]
    h1 = lax.dot_general(xb, w1_ref[0], (((1,), (1,)), ((), ())),
                         preferred_element_type=jnp.float32)
    h3 = lax.dot_general(xb, w3_ref[0], (((1,), (1,)), ((), ())),
                         preferred_element_type=jnp.float32)
    g = (h1 * jax.nn.sigmoid(h1) * h3).astype(jnp.bfloat16)
    ys_ref[...] = lax.dot_general(g, w2_ref[0], (((1,), (1,)), ((), ())),
                                  preferred_element_type=jnp.float32)


def _gemm(be, xr, xs, w1, w3, w2):
    grid_spec = pltpu.PrefetchScalarGridSpec(
        num_scalar_prefetch=2,
        grid=(NBMAX,),
        in_specs=[
            pl.BlockSpec((BLK, D), lambda i, be, xr: (xr[i], 0)),
            pl.BlockSpec((1, F, D), lambda i, be, xr: (be[i], 0, 0)),
            pl.BlockSpec((1, F, D), lambda i, be, xr: (be[i], 0, 0)),
            pl.BlockSpec((1, D, F), lambda i, be, xr: (be[i], 0, 0)),
        ],
        out_specs=pl.BlockSpec((BLK, D), lambda i, be, xr: (xr[i], 0)),
    )
    return pl.pallas_call(
        _gemm_body,
        grid_spec=grid_spec,
        out_shape=jax.ShapeDtypeStruct((NPAD, D), jnp.float32),
    )(be, xr, xs, w1, w3, w2)


def _shared_body(x_ref, sw1_ref, sw3_ref, sw2_ref, r0_ref, r1_ref, wc_ref, o_ref):
    xb = x_ref[...]
    h1 = lax.dot_general(xb, sw1_ref[...], (((1,), (1,)), ((), ())),
                         preferred_element_type=jnp.float32)
    h3 = lax.dot_general(xb, sw3_ref[...], (((1,), (1,)), ((), ())),
                         preferred_element_type=jnp.float32)
    g = (h1 * jax.nn.sigmoid(h1) * h3).astype(jnp.bfloat16)
    y = lax.dot_general(g, sw2_ref[...], (((1,), (1,)), ((), ())),
                        preferred_element_type=jnp.float32)
    o_ref[...] = (y + wc_ref[:, 0:1] * r0_ref[...]
                  + wc_ref[:, 1:2] * r1_ref[...])


def _shared(x, sw1, sw3, sw2, r0, r1, wcol):
    nblk = T // SBLK
    return pl.pallas_call(
        _shared_body,
        grid=(nblk,),
        in_specs=[
            pl.BlockSpec((SBLK, D), lambda i: (i, 0)),
            pl.BlockSpec((SF, D), lambda i: (0, 0)),
            pl.BlockSpec((SF, D), lambda i: (0, 0)),
            pl.BlockSpec((D, SF), lambda i: (0, 0)),
            pl.BlockSpec((SBLK, D), lambda i: (i, 0)),
            pl.BlockSpec((SBLK, D), lambda i: (i, 0)),
            pl.BlockSpec((SBLK, 128), lambda i: (i, 0)),
        ],
        out_specs=pl.BlockSpec((SBLK, D), lambda i: (i, 0)),
        out_shape=jax.ShapeDtypeStruct((T, D), jnp.float32),
    )(x, sw1, sw3, sw2, r0, r1, wcol)


def kernel(hidden_states, gate_weight, gate_bias, w1, w3, w2, sw1, sw3, sw2):
    shape = hidden_states.shape
    x = hidden_states.reshape(-1, D)

    logits = x @ gate_weight.T
    scores = jax.nn.sigmoid(logits)
    scores_for_topk = scores + gate_bias
    scores_view = scores_for_topk.reshape(T, G, -1)
    group_scores = lax.top_k(scores_view, 2)[0].sum(axis=-1)
    group_idx = lax.top_k(group_scores, 2)[1]
    gi = jnp.arange(G)[None, :]
    mask = (gi != group_idx[:, 0:1]) & (gi != group_idx[:, 1:2])
    masked = jnp.where(mask[:, :, None], -jnp.inf, scores_view).reshape(T, E)
    _, indices = lax.top_k(masked, K)
    weights = jnp.take_along_axis(scores, indices, axis=1)
    weights = weights / (weights.sum(axis=-1, keepdims=True) + 1e-6)
    weights = weights * SCALE

    wcol, dcol, meta = _meta(indices.astype(jnp.int32), weights)
    be = meta[0, :NBMAX]
    xr = meta[1, :NBMAX]
    d0 = dcol[:, 0]
    d1 = dcol[:, 1]

    xs = _dispatch(x, d0, d1)
    ys = _gemm(be, xr, xs, w1, w3, w2)
    r0, r1 = _gather(ys, d0, d1)
    out = _shared(x, sw1, sw3, sw2, r0, r1, wcol)
    return out.reshape(shape)
